# baseline (speedup 1.0000x reference)
import jax
import jax.numpy as jnp
from jax import lax
from jax.experimental import pallas as pl
from jax.experimental.pallas import tpu as pltpu
from jax.experimental.pallas import tpu_sc as plsc

DEPTH = 16
SIZES = (1024, 512, 256, 128)
LEVELS = len(SIZES)
BATCH = 4
HO = 512
WO = 512
HW = HO * WO
NPIX = BATCH * HW

_BASES = []
_off = 0
for _s in SIZES:
    _BASES.append(_off)
    _off += _s * _s
ZERO_ROW = _off
TABLE_ROWS = _off + 1

NC = 2
NS = 16
NW = NC * NS
LANES = 16

P = 128
NCL = 4 * LEVELS
NHALF = NPIX // 2
PIX_PER_W = NHALF // NW
BLOCKS = PIX_PER_W // P
GROUPS = P // LANES


def _texture_blend_kernel(xs_hbm, ys_hbm, table_hbm, out_hbm,
                          xv, yv, idx_ref, txs_ref, tys_ref, rows_ref,
                          outb_ref, sin0, sin1, sg0, sg1, so0, so1):
    wid = lax.axis_index("s") * NC + lax.axis_index("c")
    iota16 = lax.iota(jnp.int32, LANES)
    sem_in = (sin0, sin1)
    sem_g = (sg0, sg1)
    sem_out = (so0, so1)
    pbase = wid * PIX_PER_W

    def fire_in(t, b):
        p0 = pbase + t * P
        pltpu.async_copy(xs_hbm.at[pl.ds(p0, P)], xv.at[b], sem_in[b])
        pltpu.async_copy(ys_hbm.at[pl.ds(p0, P)], yv.at[b], sem_in[b])

    def wait_in(b):
        pltpu.make_async_copy(xs_hbm.at[pl.ds(0, P)], xv.at[b], sem_in[b]).wait()
        pltpu.make_async_copy(ys_hbm.at[pl.ds(0, P)], yv.at[b], sem_in[b]).wait()

    def compute(b):
        @plsc.parallel_loop(0, GROUPS)
        def cgroup(g):
            off = g * LANES
            u = xv[b, pl.ds(off, LANES)]
            v = yv[b, pl.ds(off, LANES)]
            for l in range(LEVELS):
                s = SIZES[l]
                xs_ = u * float(s) + 0.5
                ys_ = v * float(s) + 0.5
                tx_i = xs_.astype(jnp.int32)
                ty_i = ys_.astype(jnp.int32)
                tx = xs_ - tx_i.astype(jnp.float32)
                ty = ys_ - ty_i.astype(jnp.float32)
                ix0 = tx_i - 1
                iy0 = ty_i - 1
                mx0 = ix0 >= 0
                mx1 = ix0 <= s - 2
                my0 = iy0 >= 0
                my1 = iy0 <= s - 2
                rowb = iy0 * s + (ix0 + _BASES[l])
                z = jnp.int32(ZERO_ROW)
                j = 4 * l
                idx_ref[b, j + 0, pl.ds(off, LANES)] = jnp.where(mx0 & my0, rowb, z)
                idx_ref[b, j + 1, pl.ds(off, LANES)] = jnp.where(mx1 & my0, rowb + 1, z)
                idx_ref[b, j + 2, pl.ds(off, LANES)] = jnp.where(mx0 & my1, rowb + s, z)
                idx_ref[b, j + 3, pl.ds(off, LANES)] = jnp.where(mx1 & my1, rowb + (s + 1), z)
                txs_ref[b, l, pl.ds(off, LANES)] = tx
                tys_ref[b, l, pl.ds(off, LANES)] = ty

    def fire_gathers(b):
        for j in range(NCL):
            pltpu.async_copy(table_hbm.at[idx_ref.at[b, j]],
                             rows_ref.at[b, pl.ds(j * P, P)], sem_g[b])

    def wait_gathers(b):
        pltpu.make_async_copy(table_hbm.at[pl.ds(0, NCL * P)],
                              rows_ref.at[b], sem_g[b]).wait()

    def blend(b):
        @plsc.parallel_loop(0, GROUPS)
        def bgroup(g):
            off = g * LANES
            txv = [txs_ref[b, l, pl.ds(off, LANES)] for l in range(LEVELS)]
            tyv = [tys_ref[b, l, pl.ds(off, LANES)] for l in range(LEVELS)]
            for j in range(LANES):
                p = off + j
                jj = jnp.full((LANES,), j, jnp.int32)
                acc = jnp.zeros((LANES,), jnp.float32)
                for l in range(LEVELS):
                    txb = jnp.take(txv[l], jj)
                    tyb = jnp.take(tyv[l], jj)
                    base = 4 * l * P + p
                    v00 = rows_ref[b, base, :]
                    v10 = rows_ref[b, base + P, :]
                    v01 = rows_ref[b, base + 2 * P, :]
                    v11 = rows_ref[b, base + 3 * P, :]
                    top = v00 + txb * (v10 - v00)
                    bot = v01 + txb * (v11 - v01)
                    acc = acc + (top + tyb * (bot - top))
                outb_ref[b, pl.ds(p * DEPTH, DEPTH)] = acc

    def fire_out(t, b):
        p0 = pbase + t * P
        pltpu.async_copy(outb_ref.at[b], out_hbm.at[pl.ds(p0 * DEPTH, P * DEPTH)],
                         sem_out[b])

    def wait_out(b):
        pltpu.make_async_copy(outb_ref.at[b], out_hbm.at[pl.ds(0, P * DEPTH)],
                              sem_out[b]).wait()

    fire_in(0, 0)
    wait_in(0)
    compute(0)
    fire_gathers(0)
    fire_in(1, 1)

    def pair_body(g, carry):
        t = 2 * g
        wait_in(1)
        compute(1)
        fire_gathers(1)

        @pl.when(t + 2 < BLOCKS)
        def _():
            fire_in(t + 2, 0)

        @pl.when(t >= 2)
        def _():
            wait_out(0)

        wait_gathers(0)
        blend(0)
        fire_out(t, 0)

        @pl.when(t + 2 < BLOCKS)
        def _():
            wait_in(0)
            compute(0)
            fire_gathers(0)

        @pl.when(t + 3 < BLOCKS)
        def _():
            fire_in(t + 3, 1)

        @pl.when(t >= 2)
        def _():
            wait_out(1)

        wait_gathers(1)
        blend(1)
        fire_out(t + 1, 1)
        return carry

    lax.fori_loop(0, BLOCKS // 2, pair_body, 0)
    wait_out(0)
    wait_out(1)


@jax.jit
def _run(xs, ys, table):
    mesh = plsc.VectorSubcoreMesh(core_axis_name="c", subcore_axis_name="s",
                                  num_cores=NC, num_subcores=NS)
    f = pl.kernel(
        _texture_blend_kernel,
        out_type=jax.ShapeDtypeStruct((NHALF * DEPTH,), jnp.float32),
        mesh=mesh,
        compiler_params=pltpu.CompilerParams(use_tc_tiling_on_sc=False),
        scratch_types=[
            pltpu.VMEM((2, P), jnp.float32),
            pltpu.VMEM((2, P), jnp.float32),
            pltpu.VMEM((2, NCL, P), jnp.int32),
            pltpu.VMEM((2, LEVELS, P), jnp.float32),
            pltpu.VMEM((2, LEVELS, P), jnp.float32),
            pltpu.VMEM((2, NCL * P, DEPTH), jnp.float32),
            pltpu.VMEM((2, P * DEPTH), jnp.float32),
            pltpu.SemaphoreType.DMA,
            pltpu.SemaphoreType.DMA,
            pltpu.SemaphoreType.DMA,
            pltpu.SemaphoreType.DMA,
            pltpu.SemaphoreType.DMA,
            pltpu.SemaphoreType.DMA,
        ],
    )
    return f(xs, ys, table)


def kernel(input, tex0, tex1, tex2, tex3):
    parts = [jnp.transpose(t, (1, 2, 0)).reshape(-1, DEPTH)
             for t in (tex0, tex1, tex2, tex3)]
    parts.append(jnp.zeros((1, DEPTH), jnp.float32))
    table = jnp.concatenate(parts, axis=0)
    halves = []
    for h in range(2):
        sl = input[2 * h:2 * h + 2]
        xs = sl[..., 0].reshape(NHALF)
        ys = sl[..., 1].reshape(NHALF)
        o = _run(xs, ys, table).reshape(2, HO * WO, DEPTH)
        halves.append(jnp.transpose(o, (0, 2, 1)))
    out = jnp.concatenate(halves, axis=0)
    return out.reshape(BATCH, DEPTH, HO, WO)


# speedup vs baseline: 1.0208x; 1.0208x over previous
import jax
import jax.numpy as jnp
from jax import lax
from jax.experimental import pallas as pl
from jax.experimental.pallas import tpu as pltpu
from jax.experimental.pallas import tpu_sc as plsc

DEPTH = 16
SIZES = (1024, 512, 256, 128)
LEVELS = len(SIZES)
BATCH = 4
HO = 512
WO = 512
HW = HO * WO
NPIX = BATCH * HW

_BASES = []
_off = 0
for _s in SIZES:
    _BASES.append(_off)
    _off += _s * _s
ZERO_ROW = _off
TABLE_ROWS = _off + 1

NC = 2
NS = 16
NW = NC * NS
LANES = 16

P = 128
NCL = 4 * LEVELS
PIX_PER_W = NPIX // NW
BLOCKS = PIX_PER_W // P
GROUPS = P // LANES
W_PER_IMG = HW // PIX_PER_W


def _texture_blend_kernel(xs_hbm, ys_hbm, table_hbm, out_hbm,
                          xv, yv, idx_ref, txs_ref, tys_ref, rows_ref,
                          outb_ref, sin0, sin1, sg0, sg1, so0, so1):
    wid = lax.axis_index("s") * NC + lax.axis_index("c")
    iota16 = lax.iota(jnp.int32, LANES)
    sem_in = (sin0, sin1)
    sem_g = (sg0, sg1)
    sem_out = (so0, so1)
    pbase = wid * PIX_PER_W

    def fire_in(t, b):
        p0 = pbase + t * P
        pltpu.async_copy(xs_hbm.at[pl.ds(p0, P)], xv.at[b], sem_in[b])
        pltpu.async_copy(ys_hbm.at[pl.ds(p0, P)], yv.at[b], sem_in[b])

    def wait_in(b):
        pltpu.make_async_copy(xs_hbm.at[pl.ds(0, P)], xv.at[b], sem_in[b]).wait()
        pltpu.make_async_copy(ys_hbm.at[pl.ds(0, P)], yv.at[b], sem_in[b]).wait()

    def compute(b):
        @plsc.parallel_loop(0, GROUPS)
        def cgroup(g):
            off = g * LANES
            u = xv[b, pl.ds(off, LANES)]
            v = yv[b, pl.ds(off, LANES)]
            for l in range(LEVELS):
                s = SIZES[l]
                xs_ = u * float(s) + 0.5
                ys_ = v * float(s) + 0.5
                tx_i = xs_.astype(jnp.int32)
                ty_i = ys_.astype(jnp.int32)
                tx = xs_ - tx_i.astype(jnp.float32)
                ty = ys_ - ty_i.astype(jnp.float32)
                ix0 = tx_i - 1
                iy0 = ty_i - 1
                mx0 = ix0 >= 0
                mx1 = ix0 <= s - 2
                my0 = iy0 >= 0
                my1 = iy0 <= s - 2
                rowb = iy0 * s + (ix0 + _BASES[l])
                z = jnp.int32(ZERO_ROW)
                j = 4 * l
                idx_ref[b, j + 0, pl.ds(off, LANES)] = jnp.where(mx0 & my0, rowb, z)
                idx_ref[b, j + 1, pl.ds(off, LANES)] = jnp.where(mx1 & my0, rowb + 1, z)
                idx_ref[b, j + 2, pl.ds(off, LANES)] = jnp.where(mx0 & my1, rowb + s, z)
                idx_ref[b, j + 3, pl.ds(off, LANES)] = jnp.where(mx1 & my1, rowb + (s + 1), z)
                txs_ref[b, l, pl.ds(off, LANES)] = tx
                tys_ref[b, l, pl.ds(off, LANES)] = ty

    def fire_gathers(b):
        for j in range(NCL):
            pltpu.async_copy(table_hbm.at[idx_ref.at[b, j]],
                             rows_ref.at[b, pl.ds(j * P, P)], sem_g[b])

    def wait_gathers(b):
        pltpu.make_async_copy(table_hbm.at[pl.ds(0, NCL * P)],
                              rows_ref.at[b], sem_g[b]).wait()

    def blend(b):
        @plsc.parallel_loop(0, GROUPS, unroll=2)
        def bgroup(g):
            off = g * LANES
            txv = [txs_ref[b, l, pl.ds(off, LANES)] for l in range(LEVELS)]
            tyv = [tys_ref[b, l, pl.ds(off, LANES)] for l in range(LEVELS)]
            for j in range(LANES):
                p = off + j
                jj = jnp.full((LANES,), j, jnp.int32)
                acc = jnp.zeros((LANES,), jnp.float32)
                for l in range(LEVELS):
                    txb = jnp.take(txv[l], jj)
                    tyb = jnp.take(tyv[l], jj)
                    base = 4 * l * P + p
                    v00 = rows_ref[b, base, :]
                    v10 = rows_ref[b, base + P, :]
                    v01 = rows_ref[b, base + 2 * P, :]
                    v11 = rows_ref[b, base + 3 * P, :]
                    top = v00 + txb * (v10 - v00)
                    bot = v01 + txb * (v11 - v01)
                    acc = acc + (top + tyb * (bot - top))
                outb_ref[b, pl.ds(p * DEPTH, DEPTH)] = acc

    def fire_out(t, b):
        p0 = pbase + t * P
        pltpu.async_copy(outb_ref.at[b], out_hbm.at[pl.ds(p0 * DEPTH, P * DEPTH)],
                         sem_out[b])

    def wait_out(b):
        pltpu.make_async_copy(outb_ref.at[b], out_hbm.at[pl.ds(0, P * DEPTH)],
                              sem_out[b]).wait()

    fire_in(0, 0)
    wait_in(0)
    compute(0)
    fire_gathers(0)
    fire_in(1, 1)

    def pair_body(g, carry):
        t = 2 * g
        wait_in(1)
        compute(1)
        fire_gathers(1)

        @pl.when(t + 2 < BLOCKS)
        def _():
            fire_in(t + 2, 0)

        @pl.when(t >= 2)
        def _():
            wait_out(0)

        wait_gathers(0)
        blend(0)
        fire_out(t, 0)

        @pl.when(t + 2 < BLOCKS)
        def _():
            wait_in(0)
            compute(0)
            fire_gathers(0)

        @pl.when(t + 3 < BLOCKS)
        def _():
            fire_in(t + 3, 1)

        @pl.when(t >= 2)
        def _():
            wait_out(1)

        wait_gathers(1)
        blend(1)
        fire_out(t + 1, 1)
        return carry

    lax.fori_loop(0, BLOCKS // 2, pair_body, 0)
    wait_out(0)
    wait_out(1)


@jax.jit
def _run(xs, ys, table):
    mesh = plsc.VectorSubcoreMesh(core_axis_name="c", subcore_axis_name="s",
                                  num_cores=NC, num_subcores=NS)
    f = pl.kernel(
        _texture_blend_kernel,
        out_type=jax.ShapeDtypeStruct((NPIX * DEPTH,), jnp.float32),
        mesh=mesh,
        compiler_params=pltpu.CompilerParams(use_tc_tiling_on_sc=False),
        scratch_types=[
            pltpu.VMEM((2, P), jnp.float32),
            pltpu.VMEM((2, P), jnp.float32),
            pltpu.VMEM((2, NCL, P), jnp.int32),
            pltpu.VMEM((2, LEVELS, P), jnp.float32),
            pltpu.VMEM((2, LEVELS, P), jnp.float32),
            pltpu.VMEM((2, NCL * P, DEPTH), jnp.float32),
            pltpu.VMEM((2, P * DEPTH), jnp.float32),
            pltpu.SemaphoreType.DMA,
            pltpu.SemaphoreType.DMA,
            pltpu.SemaphoreType.DMA,
            pltpu.SemaphoreType.DMA,
            pltpu.SemaphoreType.DMA,
            pltpu.SemaphoreType.DMA,
        ],
    )
    return f(xs, ys, table)


def kernel(input, tex0, tex1, tex2, tex3):
    xs = input[..., 0].reshape(NPIX)
    ys = input[..., 1].reshape(NPIX)
    parts = [jnp.transpose(t, (1, 2, 0)).reshape(-1, DEPTH)
             for t in (tex0, tex1, tex2, tex3)]
    parts.append(jnp.zeros((1, DEPTH), jnp.float32))
    table = jnp.concatenate(parts, axis=0)
    out = _run(xs, ys, table)
    out = out.reshape(BATCH, HO * WO, DEPTH)
    return jnp.transpose(out, (0, 2, 1)).reshape(BATCH, DEPTH, HO, WO)


# speedup vs baseline: 1.0224x; 1.0015x over previous
import jax
import jax.numpy as jnp
from jax import lax
from jax.experimental import pallas as pl
from jax.experimental.pallas import tpu as pltpu
from jax.experimental.pallas import tpu_sc as plsc

DEPTH = 16
SIZES = (1024, 512, 256, 128)
LEVELS = len(SIZES)
BATCH = 4
HO = 512
WO = 512
HW = HO * WO
NPIX = BATCH * HW

_BASES = []
_off = 0
for _s in SIZES:
    _BASES.append(_off)
    _off += _s * _s
ZERO_ROW = _off
TABLE_ROWS = _off + 1

NC = 2
NS = 16
NW = NC * NS
LANES = 16

P = 128
NCL = 4 * LEVELS
PIX_PER_W = NPIX // NW
BLOCKS = PIX_PER_W // P
GROUPS = P // LANES
W_PER_IMG = HW // PIX_PER_W


def _texture_blend_kernel(xs_hbm, ys_hbm, table_hbm, out_hbm,
                          xv, yv, idx_ref, txs_ref, tys_ref, rows_ref,
                          outb_ref, sin0, sin1, sg0, sg1, so0, so1):
    wid = lax.axis_index("s") * NC + lax.axis_index("c")
    iota16 = lax.iota(jnp.int32, LANES)
    sem_in = (sin0, sin1)
    sem_g = (sg0, sg1)
    sem_out = (so0, so1)
    pbase = wid * PIX_PER_W

    def fire_in(t, b):
        p0 = pbase + t * P
        pltpu.async_copy(xs_hbm.at[pl.ds(p0, P)], xv.at[b], sem_in[b])
        pltpu.async_copy(ys_hbm.at[pl.ds(p0, P)], yv.at[b], sem_in[b])

    def wait_in(b):
        pltpu.make_async_copy(xs_hbm.at[pl.ds(0, P)], xv.at[b], sem_in[b]).wait()
        pltpu.make_async_copy(ys_hbm.at[pl.ds(0, P)], yv.at[b], sem_in[b]).wait()

    def compute(b):
        @plsc.parallel_loop(0, GROUPS)
        def cgroup(g):
            off = g * LANES
            u = xv[b, pl.ds(off, LANES)]
            v = yv[b, pl.ds(off, LANES)]
            for l in range(LEVELS):
                s = SIZES[l]
                xs_ = u * float(s) + 0.5
                ys_ = v * float(s) + 0.5
                tx_i = xs_.astype(jnp.int32)
                ty_i = ys_.astype(jnp.int32)
                tx = xs_ - tx_i.astype(jnp.float32)
                ty = ys_ - ty_i.astype(jnp.float32)
                ix0 = tx_i - 1
                iy0 = ty_i - 1
                mx0 = ix0 >= 0
                mx1 = ix0 <= s - 2
                my0 = iy0 >= 0
                my1 = iy0 <= s - 2
                rowb = iy0 * s + (ix0 + _BASES[l])
                z = jnp.int32(ZERO_ROW)
                j = 4 * l
                idx_ref[b, j + 0, pl.ds(off, LANES)] = jnp.where(mx0 & my0, rowb, z)
                idx_ref[b, j + 1, pl.ds(off, LANES)] = jnp.where(mx1 & my0, rowb + 1, z)
                idx_ref[b, j + 2, pl.ds(off, LANES)] = jnp.where(mx0 & my1, rowb + s, z)
                idx_ref[b, j + 3, pl.ds(off, LANES)] = jnp.where(mx1 & my1, rowb + (s + 1), z)
                txs_ref[b, l, pl.ds(off, LANES)] = tx
                tys_ref[b, l, pl.ds(off, LANES)] = ty

    def fire_gathers(b):
        for j in range(NCL):
            pltpu.async_copy(table_hbm.at[idx_ref.at[b, j]],
                             rows_ref.at[b, pl.ds(j * P, P)], sem_g[b])

    def wait_gathers(b):
        pltpu.make_async_copy(table_hbm.at[pl.ds(0, NCL * P)],
                              rows_ref.at[b], sem_g[b]).wait()

    def blend(b):
        @plsc.parallel_loop(0, GROUPS)
        def bgroup(g):
            off = g * LANES
            txv = [txs_ref[b, l, pl.ds(off, LANES)] for l in range(LEVELS)]
            tyv = [tys_ref[b, l, pl.ds(off, LANES)] for l in range(LEVELS)]
            for j in range(LANES):
                p = off + j
                jj = jnp.full((LANES,), j, jnp.int32)
                acc = jnp.zeros((LANES,), jnp.float32)
                for l in range(LEVELS):
                    txb = jnp.take(txv[l], jj)
                    tyb = jnp.take(tyv[l], jj)
                    base = 4 * l * P + p
                    v00 = rows_ref[b, base, :]
                    v10 = rows_ref[b, base + P, :]
                    v01 = rows_ref[b, base + 2 * P, :]
                    v11 = rows_ref[b, base + 3 * P, :]
                    top = v00 + txb * (v10 - v00)
                    bot = v01 + txb * (v11 - v01)
                    acc = acc + (top + tyb * (bot - top))
                outb_ref[b, pl.ds(p * DEPTH, DEPTH)] = acc

    def fire_out(t, b):
        p0 = pbase + t * P
        pltpu.async_copy(outb_ref.at[b], out_hbm.at[pl.ds(p0 * DEPTH, P * DEPTH)],
                         sem_out[b])

    def wait_out(b):
        pltpu.make_async_copy(outb_ref.at[b], out_hbm.at[pl.ds(0, P * DEPTH)],
                              sem_out[b]).wait()

    fire_in(0, 0)
    wait_in(0)
    compute(0)
    fire_gathers(0)
    fire_in(1, 1)

    def pair_body(g, carry):
        t = 2 * g
        wait_in(1)
        compute(1)
        fire_gathers(1)

        @pl.when(t + 2 < BLOCKS)
        def _():
            fire_in(t + 2, 0)

        @pl.when(t >= 2)
        def _():
            wait_out(0)

        wait_gathers(0)
        blend(0)
        fire_out(t, 0)

        @pl.when(t + 2 < BLOCKS)
        def _():
            wait_in(0)
            compute(0)
            fire_gathers(0)

        @pl.when(t + 3 < BLOCKS)
        def _():
            fire_in(t + 3, 1)

        @pl.when(t >= 2)
        def _():
            wait_out(1)

        wait_gathers(1)
        blend(1)
        fire_out(t + 1, 1)
        return carry

    lax.fori_loop(0, BLOCKS // 2, pair_body, 0)
    wait_out(0)
    wait_out(1)


@jax.jit
def _run(xs, ys, table):
    mesh = plsc.VectorSubcoreMesh(core_axis_name="c", subcore_axis_name="s",
                                  num_cores=NC, num_subcores=NS)
    f = pl.kernel(
        _texture_blend_kernel,
        out_type=jax.ShapeDtypeStruct((NPIX * DEPTH,), jnp.float32),
        mesh=mesh,
        compiler_params=pltpu.CompilerParams(use_tc_tiling_on_sc=False),
        scratch_types=[
            pltpu.VMEM((2, P), jnp.float32),
            pltpu.VMEM((2, P), jnp.float32),
            pltpu.VMEM((2, NCL, P), jnp.int32),
            pltpu.VMEM((2, LEVELS, P), jnp.float32),
            pltpu.VMEM((2, LEVELS, P), jnp.float32),
            pltpu.VMEM((2, NCL * P, DEPTH), jnp.float32),
            pltpu.VMEM((2, P * DEPTH), jnp.float32),
            pltpu.SemaphoreType.DMA,
            pltpu.SemaphoreType.DMA,
            pltpu.SemaphoreType.DMA,
            pltpu.SemaphoreType.DMA,
            pltpu.SemaphoreType.DMA,
            pltpu.SemaphoreType.DMA,
        ],
    )
    return f(xs, ys, table)


def kernel(input, tex0, tex1, tex2, tex3):
    xs = input[..., 0].reshape(NPIX)
    ys = input[..., 1].reshape(NPIX)
    parts = [jnp.transpose(t, (1, 2, 0)).reshape(-1, DEPTH)
             for t in (tex0, tex1, tex2, tex3)]
    parts.append(jnp.zeros((1, DEPTH), jnp.float32))
    table = jnp.concatenate(parts, axis=0)
    out = _run(xs, ys, table)
    out = out.reshape(BATCH, HO * WO, DEPTH)
    return jnp.transpose(out, (0, 2, 1)).reshape(BATCH, DEPTH, HO, WO)
